# baseline (speedup 1.0000x reference)
import jax
import jax.numpy as jnp
from jax.experimental import pallas as pl


_ROWS = 16384
_COLS = 4096
_BI = 512
_BJ = 512


def _onehot_kernel(perm_ref, x_ref, out_ref):
    j = pl.program_id(1)
    pv = perm_ref[:, pl.ds(j * _BJ, _BJ)]
    k_iota = jax.lax.broadcasted_iota(jnp.int32, (_COLS, _BJ), 0)
    p = (k_iota == pv).astype(jnp.bfloat16)
    xb = x_ref[...]
    hi = xb.astype(jnp.bfloat16)
    lo = (xb - hi.astype(jnp.float32)).astype(jnp.bfloat16)
    acc = jnp.dot(hi, p, preferred_element_type=jnp.float32)
    acc += jnp.dot(lo, p, preferred_element_type=jnp.float32)
    out_ref[...] = acc


def kernel(x, perm):
    perm2 = perm.astype(jnp.int32).reshape(1, _COLS)
    out = pl.pallas_call(
        _onehot_kernel,
        grid=(_ROWS // _BI, _COLS // _BJ),
        in_specs=[
            pl.BlockSpec((1, _COLS), lambda i, j: (0, 0)),
            pl.BlockSpec((_BI, _COLS), lambda i, j: (i, 0)),
        ],
        out_specs=pl.BlockSpec((_BI, _BJ), lambda i, j: (i, j)),
        out_shape=jax.ShapeDtypeStruct((_ROWS, _COLS), jnp.float32),
    )(perm2, x)
    return (out, 0)


# speedup vs baseline: 1.7004x; 1.7004x over previous
import jax
import jax.numpy as jnp
from jax import lax
from jax.experimental import pallas as pl
from jax.experimental.pallas import tpu as pltpu
from jax.experimental.pallas import tpu_sc as plsc


_ROWS = 16384
_COLS = 4096
_HALF = _COLS // 2
_NC = 2
_NS = 16
_NW = _NC * _NS
_SLABS = _ROWS // 8
_SLABS_PER_W = _SLABS // _NW


def _sc_body(x_hbm, perm_hbm, o_hbm, perm_v, in_v, out_v,
             sem_in, sem_out):
    wid = lax.axis_index("s") * _NC + lax.axis_index("c")
    base = wid * _SLABS_PER_W

    pltpu.sync_copy(perm_hbm, perm_v)

    def rows(g):
        return pl.ds((base + g) * 8, 8)

    pltpu.async_copy(x_hbm.at[rows(0), :], in_v.at[0], sem_in.at[0])
    pltpu.async_copy(x_hbm.at[rows(1), :], in_v.at[1], sem_in.at[1])

    @pl.loop(0, _SLABS_PER_W, step=2)
    def _(g0):
        for b in range(2):
            g = g0 + b

            pltpu.make_async_copy(
                x_hbm.at[rows(g), :], in_v.at[b], sem_in.at[b]
            ).wait()

            for h in range(2):
                @pl.when(g >= 1)
                def _():
                    pltpu.make_async_copy(
                        out_v.at[h],
                        o_hbm.at[rows(g - 1), pl.ds(h * _HALF, _HALF)],
                        sem_out.at[h],
                    ).wait()

                @pl.loop(0, _HALF // 16)
                def _(t):
                    j0 = pl.multiple_of(t * 16, 16)
                    cvec = perm_v[pl.ds(h * _HALF + j0, 16)]
                    for s in range(8):
                        svec = jnp.full((16,), s, jnp.int32)
                        vals = plsc.load_gather(in_v.at[b], [svec, cvec])
                        out_v[h, s, pl.ds(j0, 16)] = vals

                pltpu.async_copy(
                    out_v.at[h],
                    o_hbm.at[rows(g), pl.ds(h * _HALF, _HALF)],
                    sem_out.at[h],
                )

            @pl.when(g + 2 < _SLABS_PER_W)
            def _():
                pltpu.async_copy(
                    x_hbm.at[rows(g + 2), :], in_v.at[b], sem_in.at[b]
                )

    g_last = _SLABS_PER_W - 1
    for h in range(2):
        pltpu.make_async_copy(
            out_v.at[h],
            o_hbm.at[rows(g_last), pl.ds(h * _HALF, _HALF)],
            sem_out.at[h],
        ).wait()


def kernel(x, perm):
    perm32 = perm.astype(jnp.int32)
    mesh = plsc.VectorSubcoreMesh(core_axis_name="c", subcore_axis_name="s")
    k = pl.kernel(
        _sc_body,
        mesh=mesh,
        compiler_params=pltpu.CompilerParams(needs_layout_passes=False),
        out_type=jax.ShapeDtypeStruct((_ROWS, _COLS), jnp.float32),
        scratch_types=[
            pltpu.VMEM((_COLS,), jnp.int32),
            pltpu.VMEM((2, 8, _COLS), jnp.float32),
            pltpu.VMEM((2, 8, _HALF), jnp.float32),
            pltpu.SemaphoreType.DMA((2,)),
            pltpu.SemaphoreType.DMA((2,)),
        ],
    )
    out = k(x, perm32)
    return (out, 0)


# speedup vs baseline: 5.5914x; 3.2883x over previous
import jax
import jax.numpy as jnp
from jax import lax
from jax.experimental import pallas as pl
from jax.experimental.pallas import tpu as pltpu
from jax.experimental.pallas import tpu_sc as plsc


_ROWS = 16384
_COLS = 4096
_HALF = _COLS // 2
_NC = 2
_NS = 16
_NW = _NC * _NS
_SLABS = _ROWS // 8
_SLABS_PER_W = _SLABS // _NW


def _sc_body(x_hbm, perm_hbm, o_hbm, perm_v, in_v, out_v,
             sem_in, sem_out):
    wid = lax.axis_index("s") * _NC + lax.axis_index("c")
    base = wid * _SLABS_PER_W

    pltpu.sync_copy(perm_hbm, perm_v)

    def rows(g):
        return pl.ds((base + g) * 8, 8)

    pltpu.async_copy(x_hbm.at[rows(0), :], in_v.at[0], sem_in.at[0])
    pltpu.async_copy(x_hbm.at[rows(1), :], in_v.at[1], sem_in.at[1])

    @pl.loop(0, _SLABS_PER_W, step=2)
    def _(g0):
        for b in range(2):
            g = g0 + b

            pltpu.make_async_copy(
                x_hbm.at[rows(g), :], in_v.at[b], sem_in.at[b]
            ).wait()

            for h in range(2):
                @pl.when(g >= 1)
                def _():
                    pltpu.make_async_copy(
                        out_v.at[h],
                        o_hbm.at[rows(g - 1), pl.ds(h * _HALF, _HALF)],
                        sem_out.at[h],
                    ).wait()

                @plsc.parallel_loop(0, _HALF // 16, step=1, unroll=4)
                def _(t):
                    j0 = pl.multiple_of(t * 16, 16)
                    cvec = perm_v[pl.ds(h * _HALF + j0, 16)]
                    for s in range(8):
                        svec = jnp.full((16,), s, jnp.int32)
                        vals = plsc.load_gather(in_v.at[b], [svec, cvec])
                        out_v[h, s, pl.ds(j0, 16)] = vals

                pltpu.async_copy(
                    out_v.at[h],
                    o_hbm.at[rows(g), pl.ds(h * _HALF, _HALF)],
                    sem_out.at[h],
                )

            @pl.when(g + 2 < _SLABS_PER_W)
            def _():
                pltpu.async_copy(
                    x_hbm.at[rows(g + 2), :], in_v.at[b], sem_in.at[b]
                )

    g_last = _SLABS_PER_W - 1
    for h in range(2):
        pltpu.make_async_copy(
            out_v.at[h],
            o_hbm.at[rows(g_last), pl.ds(h * _HALF, _HALF)],
            sem_out.at[h],
        ).wait()


def kernel(x, perm):
    perm32 = perm.astype(jnp.int32)
    mesh = plsc.VectorSubcoreMesh(core_axis_name="c", subcore_axis_name="s")
    k = pl.kernel(
        _sc_body,
        mesh=mesh,
        compiler_params=pltpu.CompilerParams(needs_layout_passes=False),
        out_type=jax.ShapeDtypeStruct((_ROWS, _COLS), jnp.float32),
        scratch_types=[
            pltpu.VMEM((_COLS,), jnp.int32),
            pltpu.VMEM((2, 8, _COLS), jnp.float32),
            pltpu.VMEM((2, 8, _HALF), jnp.float32),
            pltpu.SemaphoreType.DMA((2,)),
            pltpu.SemaphoreType.DMA((2,)),
        ],
    )
    out = k(x, perm32)
    return (out, 0)


# speedup vs baseline: 5.5936x; 1.0004x over previous
import jax
import jax.numpy as jnp
from jax import lax
from jax.experimental import pallas as pl
from jax.experimental.pallas import tpu as pltpu
from jax.experimental.pallas import tpu_sc as plsc


_ROWS = 16384
_COLS = 4096
_HALF = _COLS // 2
_NC = 2
_NS = 16
_NW = _NC * _NS
_SLABS = _ROWS // 8
_SLABS_PER_W = _SLABS // _NW


def _sc_body(x_hbm, perm_hbm, o_hbm, perm_v, in_v, out_v,
             sem_in, sem_out):
    wid = lax.axis_index("s") * _NC + lax.axis_index("c")
    base = wid * _SLABS_PER_W

    pltpu.sync_copy(perm_hbm, perm_v)

    def rows(g):
        return pl.ds((base + g) * 8, 8)

    pltpu.async_copy(x_hbm.at[rows(0), :], in_v.at[0], sem_in.at[0])
    pltpu.async_copy(x_hbm.at[rows(1), :], in_v.at[1], sem_in.at[1])

    @pl.loop(0, _SLABS_PER_W, step=2)
    def _(g0):
        for b in range(2):
            g = g0 + b

            pltpu.make_async_copy(
                x_hbm.at[rows(g), :], in_v.at[b], sem_in.at[b]
            ).wait()

            for h in range(2):
                @pl.when(g >= 1)
                def _():
                    pltpu.make_async_copy(
                        out_v.at[h],
                        o_hbm.at[rows(g - 1), pl.ds(h * _HALF, _HALF)],
                        sem_out.at[h],
                    ).wait()

                @plsc.parallel_loop(0, _HALF // 16, step=1, unroll=8)
                def _(t):
                    j0 = pl.multiple_of(t * 16, 16)
                    cvec = perm_v[pl.ds(h * _HALF + j0, 16)]
                    for s in range(8):
                        svec = jnp.full((16,), s, jnp.int32)
                        vals = plsc.load_gather(in_v.at[b], [svec, cvec])
                        out_v[h, s, pl.ds(j0, 16)] = vals

                pltpu.async_copy(
                    out_v.at[h],
                    o_hbm.at[rows(g), pl.ds(h * _HALF, _HALF)],
                    sem_out.at[h],
                )

            @pl.when(g + 2 < _SLABS_PER_W)
            def _():
                pltpu.async_copy(
                    x_hbm.at[rows(g + 2), :], in_v.at[b], sem_in.at[b]
                )

    g_last = _SLABS_PER_W - 1
    for h in range(2):
        pltpu.make_async_copy(
            out_v.at[h],
            o_hbm.at[rows(g_last), pl.ds(h * _HALF, _HALF)],
            sem_out.at[h],
        ).wait()


def kernel(x, perm):
    perm32 = perm.astype(jnp.int32)
    mesh = plsc.VectorSubcoreMesh(core_axis_name="c", subcore_axis_name="s")
    k = pl.kernel(
        _sc_body,
        mesh=mesh,
        compiler_params=pltpu.CompilerParams(needs_layout_passes=False),
        out_type=jax.ShapeDtypeStruct((_ROWS, _COLS), jnp.float32),
        scratch_types=[
            pltpu.VMEM((_COLS,), jnp.int32),
            pltpu.VMEM((2, 8, _COLS), jnp.float32),
            pltpu.VMEM((2, 8, _HALF), jnp.float32),
            pltpu.SemaphoreType.DMA((2,)),
            pltpu.SemaphoreType.DMA((2,)),
        ],
    )
    out = k(x, perm32)
    return (out, 0)
